# baseline (speedup 1.0000x reference)
import functools

import jax
import jax.numpy as jnp
from jax import lax
from jax.experimental import pallas as pl
from jax.experimental.pallas import tpu as pltpu
from jax.experimental.pallas import tpu_sc as plsc

B = 16384
NC, NS = 2, 16
NW = NC * NS
BPW = B // NW
CHUNK = 128
NCHUNK = BPW // CHUNK

FEAT_P = 112
SEG_W = (32, 16, 16, 16, 16, 16)

_MESH = plsc.VectorSubcoreMesh(core_axis_name="c", subcore_axis_name="s",
                               num_cores=NC, num_subcores=NS)


def _sc_gather_body(emb_u, emb_t, emb_c, emb_g, emb_a, emb_r,
                    idx_u, idx_t, idx_c, idx_g, idx_a, idx_r,
                    out_u, out_t, out_c, out_g, out_a, out_r,
                    vi_u, vi_t, vi_c, vi_g, vi_a, vi_r,
                    r_u, r_t, r_c, r_g, r_a, r_r, sem):
    wid = lax.axis_index("s") * NC + lax.axis_index("c")
    base = wid * BPW

    tables = (emb_u, emb_t, emb_c, emb_g, emb_a, emb_r)
    idxs = (idx_u, idx_t, idx_c, idx_g, idx_a, idx_r)
    vidx = (vi_u, vi_t, vi_c, vi_g, vi_a, vi_r)
    rows = (r_u, r_t, r_c, r_g, r_a, r_r)
    outs = (out_u, out_t, out_c, out_g, out_a, out_r)

    for ih, iv in zip(idxs, vidx):
        pltpu.sync_copy(ih.at[wid], iv)

    copies = []
    for tbl, iv, rv in zip(tables, vidx, rows):
        for c in range(NCHUNK):
            copies.append(
                pltpu.async_copy(tbl.at[iv.at[c]],
                                 rv.at[pl.ds(c * CHUNK, CHUNK), :], sem))
    for cp in copies:
        cp.wait()

    for rv, out in zip(rows, outs):
        pltpu.sync_copy(rv, out.at[pl.ds(base, BPW), :])


_sc_gather = functools.partial(
    pl.kernel,
    out_type=tuple(jax.ShapeDtypeStruct((B, w), jnp.float32) for w in SEG_W),
    mesh=_MESH,
    scratch_types=(
        [pltpu.VMEM((NCHUNK, CHUNK), jnp.int32) for _ in range(6)]
        + [pltpu.VMEM((BPW, w), jnp.float32) for w in SEG_W]
        + [pltpu.SemaphoreType.DMA]
    ),
    compiler_params=pltpu.CompilerParams(use_tc_tiling_on_sc=False),
)(_sc_gather_body)


BLK = 512


def _mlp_body(f_u, f_t, f_c, f_g, f_a, f_r, ten,
              w1, w1t, b1, w2, b2, w3, b3, wp, bp, out):
    segs = (f_u, f_t, f_c, f_g, f_a, f_r)
    off = 0
    h = ten[...] * w1t[...] + b1[...]
    for seg, w in zip(segs, SEG_W):
        h = h + jnp.dot(seg[...], w1[pl.ds(off, w), :],
                        preferred_element_type=jnp.float32)
        off += w
    h = jnp.maximum(h, 0.0)
    h = jnp.dot(h, w2[...], preferred_element_type=jnp.float32) + b2[...]
    h = jnp.maximum(h, 0.0)
    h = jnp.dot(h, w3[...], preferred_element_type=jnp.float32) + b3[...]
    h = jnp.maximum(h, 0.0)
    out[...] = jnp.dot(h, wp[...], preferred_element_type=jnp.float32) + bp[...]


def _mlp(feats, ten, w1, w1t, b1, w2, b2, w3, b3, wp, bp):
    full = lambda shape: pl.BlockSpec(shape, lambda i: (0, 0))
    return pl.pallas_call(
        _mlp_body,
        grid=(B // BLK,),
        in_specs=(
            [pl.BlockSpec((BLK, w), lambda i: (i, 0)) for w in SEG_W]
            + [
                pl.BlockSpec((BLK, 1), lambda i: (i, 0)),
                full((FEAT_P, 256)),
                full((1, 256)),
                full((1, 256)),
                full((256, 128)),
                full((1, 128)),
                full((128, 64)),
                full((1, 64)),
                full((64, 64)),
                full((1, 64)),
            ]
        ),
        out_specs=pl.BlockSpec((BLK, 64), lambda i: (i, 0)),
        out_shape=jax.ShapeDtypeStruct((B, 64), jnp.float32),
    )(*feats, ten, w1, w1t, b1, w2, b2, w3, b3, wp, bp)


def kernel(CustomerCode, TownName, Cluster, GroupHeaderName, Area,
           RegionCategory, TenureYears,
           emb_user, emb_town, emb_cluster, emb_group, emb_area, emb_region,
           W1, b1, W2, b2, W3, b3, Wp, bp):
    pad16 = lambda t: jnp.pad(t, ((0, 0), (0, 16 - t.shape[1])))
    emb_c = pad16(emb_cluster)
    emb_g = pad16(emb_group)
    emb_a = pad16(emb_area)
    emb_r = pad16(emb_region)

    shape_idx = lambda ix: ix.reshape(NW, NCHUNK, CHUNK)
    feats = _sc_gather(
        emb_user, emb_town, emb_c, emb_g, emb_a, emb_r,
        shape_idx(CustomerCode), shape_idx(TownName), shape_idx(Cluster),
        shape_idx(GroupHeaderName), shape_idx(Area), shape_idx(RegionCategory))

    padw = lambda w: jnp.pad(w, ((0, 16 - w.shape[0]), (0, 0)))
    w1p = jnp.concatenate([
        W1[0:32], W1[32:48], padw(W1[48:56]), padw(W1[56:64]),
        padw(W1[64:68]), padw(W1[68:72])], axis=0)

    return _mlp(feats, TenureYears.reshape(B, 1),
                w1p, W1[72:73], b1.reshape(1, 256),
                W2, b2.reshape(1, 128), W3, b3.reshape(1, 64),
                Wp, bp.reshape(1, 64))


# speedup vs baseline: 1.0760x; 1.0760x over previous
import functools

import jax
import jax.numpy as jnp
from jax import lax
from jax.experimental import pallas as pl
from jax.experimental.pallas import tpu as pltpu
from jax.experimental.pallas import tpu_sc as plsc

B = 16384
NC, NS = 2, 16
NW = NC * NS
BPW = B // NW
CHUNK = 128
NCHUNK = BPW // CHUNK
L = 16

_MESH = plsc.VectorSubcoreMesh(core_axis_name="c", subcore_axis_name="s",
                               num_cores=NC, num_subcores=NS)


def _sc_gather_body(emb_u, emb_t, idx_u, idx_t, out_u, out_t,
                    vi_u, vi_t, rows, sem):
    wid = lax.axis_index("s") * NC + lax.axis_index("c")
    base = wid * BPW

    pltpu.sync_copy(idx_u.at[wid], vi_u)
    pltpu.sync_copy(idx_t.at[wid], vi_t)

    for r in range(NCHUNK):
        for i in range(CHUNK // L):
            s = pl.ds(i * L, L)
            vi_u[r, s] = lax.shift_right_logical(vi_u[r, s], 2)
            vi_t[r, s] = lax.shift_right_logical(vi_t[r, s], 3)

    for iv, tbl, out in ((vi_u, emb_u, out_u), (vi_t, emb_t, out_t)):
        copies = [
            pltpu.async_copy(tbl.at[iv.at[c]],
                             rows.at[pl.ds(c * CHUNK, CHUNK), :], sem)
            for c in range(NCHUNK)
        ]
        for cp in copies:
            cp.wait()
        pltpu.sync_copy(rows, out.at[pl.ds(base, BPW), :])


_sc_gather = functools.partial(
    pl.kernel,
    out_type=(jax.ShapeDtypeStruct((B, 128), jnp.float32),
              jax.ShapeDtypeStruct((B, 128), jnp.float32)),
    mesh=_MESH,
    scratch_types=(
        pltpu.VMEM((NCHUNK, CHUNK), jnp.int32),
        pltpu.VMEM((NCHUNK, CHUNK), jnp.int32),
        pltpu.VMEM((BPW, 128), jnp.float32),
        pltpu.SemaphoreType.DMA,
    ),
)(_sc_gather_body)


BLK = 512


def _onehot(idx, n):
    lanes = lax.broadcasted_iota(jnp.int32, (1, n), 1)
    return jnp.where(idx == lanes, 1.0, 0.0).astype(jnp.float32)


def _seg_mask(idx, mod, width):
    seg = lax.broadcasted_iota(jnp.int32, (1, 128), 1) // width
    return jnp.where((idx & (mod - 1)) == seg, 1.0, 0.0).astype(jnp.float32)


def _mlp_body(u128, t128, cc, tn, cl, gr, ar, rg, ten,
              w1u4, w1t8, w1c, w1g, w1a, w1r, w1ten, b1,
              ec, eg, ea, er, w2, b2, w3, b3, wp, bp, out):
    f32 = jnp.float32
    dot = functools.partial(jnp.dot, preferred_element_type=f32)

    h = ten[...] * w1ten[...] + b1[...]
    h = h + dot(u128[...] * _seg_mask(cc[...], 4, 32), w1u4[...])
    h = h + dot(t128[...] * _seg_mask(tn[...], 8, 16), w1t8[...])
    h = h + dot(dot(_onehot(cl[...], 128), ec[...]), w1c[...])
    h = h + dot(dot(_onehot(gr[...], 1024), eg[...]), w1g[...])
    h = h + dot(dot(_onehot(ar[...], 128), ea[...]), w1a[...])
    h = h + dot(dot(_onehot(rg[...], 128), er[...]), w1r[...])
    h = jnp.maximum(h, 0.0)
    h = jnp.maximum(dot(h, w2[...]) + b2[...], 0.0)
    h = jnp.maximum(dot(h, w3[...]) + b3[...], 0.0)
    out[...] = dot(h, wp[...]) + bp[...]


def _mlp(args):
    blk = lambda w: pl.BlockSpec((BLK, w), lambda i: (i, 0))
    full = lambda shape: pl.BlockSpec(shape, lambda i: (0, 0))
    return pl.pallas_call(
        _mlp_body,
        grid=(B // BLK,),
        in_specs=(
            [blk(128), blk(128)]
            + [blk(1)] * 7
            + [full((128, 256)), full((128, 256)), full((8, 256)),
               full((8, 256)), full((4, 256)), full((4, 256)),
               full((1, 256)), full((1, 256)),
               full((128, 8)), full((1024, 8)), full((128, 4)),
               full((128, 4)),
               full((256, 128)), full((1, 128)), full((128, 64)),
               full((1, 64)), full((64, 64)), full((1, 64))]
        ),
        out_specs=pl.BlockSpec((BLK, 64), lambda i: (i, 0)),
        out_shape=jax.ShapeDtypeStruct((B, 64), jnp.float32),
    )(*args)


def kernel(CustomerCode, TownName, Cluster, GroupHeaderName, Area,
           RegionCategory, TenureYears,
           emb_user, emb_town, emb_cluster, emb_group, emb_area, emb_region,
           W1, b1, W2, b2, W3, b3, Wp, bp):
    u128, t128 = _sc_gather(
        emb_user.reshape(250000, 128), emb_town.reshape(1250, 128),
        CustomerCode.reshape(NW, NCHUNK, CHUNK),
        TownName.reshape(NW, NCHUNK, CHUNK))

    col = lambda ix: ix.reshape(B, 1)
    padv = lambda t, v: jnp.pad(t, ((0, v - t.shape[0]), (0, 0)))
    args = (
        u128, t128,
        col(CustomerCode), col(TownName), col(Cluster), col(GroupHeaderName),
        col(Area), col(RegionCategory), TenureYears.reshape(B, 1),
        jnp.tile(W1[0:32], (4, 1)), jnp.tile(W1[32:48], (8, 1)),
        W1[48:56], W1[56:64], W1[64:68], W1[68:72], W1[72:73],
        b1.reshape(1, 256),
        padv(emb_cluster, 128), padv(emb_group, 1024), padv(emb_area, 128),
        padv(emb_region, 128),
        W2, b2.reshape(1, 128), W3, b3.reshape(1, 64),
        Wp, bp.reshape(1, 64),
    )
    return _mlp(args)


# speedup vs baseline: 1.0873x; 1.0105x over previous
import functools

import jax
import jax.numpy as jnp
from jax import lax
from jax.experimental import pallas as pl
from jax.experimental.pallas import tpu as pltpu
from jax.experimental.pallas import tpu_sc as plsc

B = 16384
NC, NS = 2, 16
NW = NC * NS
BPW = B // NW
CHUNK = 128
NCHUNK = BPW // CHUNK

_MESH = plsc.VectorSubcoreMesh(core_axis_name="c", subcore_axis_name="s",
                               num_cores=NC, num_subcores=NS)


def _sc_gather_body(emb_u, emb_t, idx_u, idx_t, out_u, out_t,
                    vi_u, vi_t, r_u, r_t, sem):
    wid = lax.axis_index("s") * NC + lax.axis_index("c")
    base = wid * BPW

    pltpu.sync_copy(idx_u.at[pl.ds(base, BPW)], vi_u)
    pltpu.sync_copy(idx_t.at[pl.ds(base, BPW)], vi_t)

    copies = []
    for iv, tbl, rv in ((vi_u, emb_u, r_u), (vi_t, emb_t, r_t)):
        for c in range(NCHUNK):
            copies.append(
                pltpu.async_copy(tbl.at[iv.at[pl.ds(c * CHUNK, CHUNK)]],
                                 rv.at[pl.ds(c * CHUNK, CHUNK), :], sem))
    for cp in copies:
        cp.wait()

    pltpu.sync_copy(r_u, out_u.at[pl.ds(base, BPW), :])
    pltpu.sync_copy(r_t, out_t.at[pl.ds(base, BPW), :])


_sc_gather = functools.partial(
    pl.kernel,
    out_type=(jax.ShapeDtypeStruct((B, 32), jnp.float32),
              jax.ShapeDtypeStruct((B, 16), jnp.float32)),
    mesh=_MESH,
    scratch_types=(
        pltpu.VMEM((BPW,), jnp.int32),
        pltpu.VMEM((BPW,), jnp.int32),
        pltpu.VMEM((BPW, 32), jnp.float32),
        pltpu.VMEM((BPW, 16), jnp.float32),
        pltpu.SemaphoreType.DMA,
    ),
    compiler_params=pltpu.CompilerParams(use_tc_tiling_on_sc=False),
)(_sc_gather_body)


BLK = 512


def _onehot(idx, n):
    lanes = lax.broadcasted_iota(jnp.int32, (1, n), 1)
    return jnp.where(idx == lanes, 1.0, 0.0).astype(jnp.float32)


def _mlp_body(u, t, cl, gr, ar, rg, ten,
              w1u, w1t, w1c, w1g, w1a, w1r, w1ten, b1,
              ec, eg, ea, er, w2, b2, w3, b3, wp, bp, out):
    f32 = jnp.float32
    dot = functools.partial(jnp.dot, preferred_element_type=f32)

    h = ten[...] * w1ten[...] + b1[...]
    h = h + dot(u[...], w1u[...])
    h = h + dot(t[...], w1t[...])
    h = h + dot(dot(_onehot(cl[...], 128), ec[...]), w1c[...])
    h = h + dot(dot(_onehot(gr[...], 1024), eg[...]), w1g[...])
    h = h + dot(dot(_onehot(ar[...], 128), ea[...]), w1a[...])
    h = h + dot(dot(_onehot(rg[...], 128), er[...]), w1r[...])
    h = jnp.maximum(h, 0.0)
    h = jnp.maximum(dot(h, w2[...]) + b2[...], 0.0)
    h = jnp.maximum(dot(h, w3[...]) + b3[...], 0.0)
    out[...] = dot(h, wp[...]) + bp[...]


def _mlp(args):
    blk = lambda w: pl.BlockSpec((BLK, w), lambda i: (i, 0))
    full = lambda shape: pl.BlockSpec(shape, lambda i: (0, 0))
    return pl.pallas_call(
        _mlp_body,
        grid=(B // BLK,),
        in_specs=(
            [blk(32), blk(16)]
            + [blk(1)] * 5
            + [full((32, 256)), full((16, 256)), full((8, 256)),
               full((8, 256)), full((4, 256)), full((4, 256)),
               full((1, 256)), full((1, 256)),
               full((128, 8)), full((1024, 8)), full((128, 4)),
               full((128, 4)),
               full((256, 128)), full((1, 128)), full((128, 64)),
               full((1, 64)), full((64, 64)), full((1, 64))]
        ),
        out_specs=pl.BlockSpec((BLK, 64), lambda i: (i, 0)),
        out_shape=jax.ShapeDtypeStruct((B, 64), jnp.float32),
    )(*args)


def kernel(CustomerCode, TownName, Cluster, GroupHeaderName, Area,
           RegionCategory, TenureYears,
           emb_user, emb_town, emb_cluster, emb_group, emb_area, emb_region,
           W1, b1, W2, b2, W3, b3, Wp, bp):
    u, t = _sc_gather(emb_user, emb_town, CustomerCode, TownName)

    col = lambda ix: ix.reshape(B, 1)
    padv = lambda tb, v: jnp.pad(tb, ((0, v - tb.shape[0]), (0, 0)))
    args = (
        u, t,
        col(Cluster), col(GroupHeaderName), col(Area), col(RegionCategory),
        TenureYears.reshape(B, 1),
        W1[0:32], W1[32:48], W1[48:56], W1[56:64], W1[64:68], W1[68:72],
        W1[72:73], b1.reshape(1, 256),
        padv(emb_cluster, 128), padv(emb_group, 1024), padv(emb_area, 128),
        padv(emb_region, 128),
        W2, b2.reshape(1, 128), W3, b3.reshape(1, 64),
        Wp, bp.reshape(1, 64),
    )
    return _mlp(args)
